# baseline (speedup 1.0000x reference)
import functools

import jax
import jax.numpy as jnp
from jax import lax
from jax.experimental import pallas as pl
from jax.experimental.pallas import tpu as pltpu
from jax.experimental.pallas import tpu_sc as plsc

_D = 128
_NC = 2
_NS = 16
_NW = _NC * _NS
_CHUNK = 128
_NBUF = 5


@functools.lru_cache(maxsize=None)
def _build(n_total):
    per_w = n_total // _NW
    nch = per_w // _CHUNK

    @functools.partial(
        pl.kernel,
        mesh=plsc.VectorSubcoreMesh(core_axis_name="c", subcore_axis_name="s"),
        out_type=jax.ShapeDtypeStruct((n_total, _D), jnp.float32),
        scratch_types=(
            [pltpu.VMEM((nch, _CHUNK), jnp.int32)]
            + [pltpu.VMEM((_CHUNK, _D), jnp.float32)] * _NBUF
            + [pltpu.SemaphoreType.DMA] * (2 * _NBUF)
        ),
    )
    def emb(idx_hbm, table_hbm, out_hbm, idx_v, *rest):
        bufs = rest[:_NBUF]
        gsems = rest[_NBUF:2 * _NBUF]
        ssems = rest[2 * _NBUF:]
        wid = lax.axis_index("s") * _NC + lax.axis_index("c")
        base = wid * per_w
        pltpu.sync_copy(idx_hbm.at[wid], idx_v)

        def gather(j, b):
            return pltpu.make_async_copy(
                table_hbm.at[idx_v.at[j]], bufs[b], gsems[b])

        def store(j, b):
            return pltpu.make_async_copy(
                bufs[b], out_hbm.at[pl.ds(base + j * _CHUNK, _CHUNK)],
                ssems[b])

        lag = _NBUF // 2
        ahead = _NBUF - lag

        for b in range(ahead):
            gather(b, b).start()

        def body(jj, carry):
            for b in range(_NBUF):
                j = jj * _NBUF + b
                bd = (b - lag) % _NBUF
                jd = j - lag
                jn = j + ahead

                @pl.when(jd >= 0)
                def _():
                    store(jd, bd).wait()

                @pl.when(jn < nch)
                def _():
                    gather(jn, bd).start()

                gather(j, b).wait()
                store(j, b).start()
            return carry

        lax.fori_loop(0, nch // _NBUF, body, 0)

        for k in range(lag):
            j = nch - lag + k
            store(j, j % _NBUF).wait()

    return emb


def kernel(sensor_ids, table):
    b, l = sensor_ids.shape
    n = b * l
    idx = sensor_ids.reshape(_NW, n // _NW // _CHUNK, _CHUNK).astype(jnp.int32)
    out = _build(n)(idx, table)
    return out.reshape(b, l, table.shape[1])
